# baseline (speedup 1.0000x reference)
import functools

import jax
import jax.numpy as jnp
from jax import lax
from jax.experimental import pallas as pl

EXPERT_NUM = 64


@functools.lru_cache(maxsize=None)
def _build(t: int, k: int, blocks: int = 4):
    bt = t // blocks

    def body(w_ref, ids_ref, wout_ref):
        x = w_ref[:]
        denom = jnp.maximum(x[0:1, :] + x[1:2, :], 1e-9)
        wout_ref[:] = x / denom
        tok = lax.broadcasted_iota(jnp.int32, (k, bt), 1)
        tok = tok + pl.program_id(0) * bt
        kk = lax.broadcasted_iota(jnp.int32, (k, bt), 0)
        ids_ref[:] = (k * tok + kk) & (EXPERT_NUM - 1)

    return pl.pallas_call(
        body,
        grid=(blocks,),
        in_specs=[pl.BlockSpec((k, bt), lambda i: (0, i))],
        out_specs=[
            pl.BlockSpec((k, bt), lambda i: (0, i)),
            pl.BlockSpec((k, bt), lambda i: (0, i)),
        ],
        out_shape=[
            jax.ShapeDtypeStruct((k, t), jnp.int32),
            jax.ShapeDtypeStruct((k, t), jnp.float32),
        ],
    )


def kernel(topk_ids, topk_weights):
    t, k = topk_ids.shape
    ids_t, wout_t = _build(t, k)(topk_weights.T)
    return ids_t.T, wout_t.T


# speedup vs baseline: 1.5500x; 1.5500x over previous
import functools

import jax
import jax.numpy as jnp
from jax import lax
from jax.experimental import pallas as pl

EXPERT_NUM = 64


@functools.lru_cache(maxsize=None)
def _build(t: int, k: int):
    def body(w_ref, ids_ref, wout_ref):
        x = w_ref[:]
        denom = jnp.maximum(x[0:1, :] + x[1:2, :], 1e-9)
        wout_ref[:] = x / denom
        tok = lax.broadcasted_iota(jnp.int32, (k, t), 1)
        kk = lax.broadcasted_iota(jnp.int32, (k, t), 0)
        ids_ref[:] = (k * tok + kk) & (EXPERT_NUM - 1)

    return pl.pallas_call(
        body,
        out_shape=[
            jax.ShapeDtypeStruct((k, t), jnp.int32),
            jax.ShapeDtypeStruct((k, t), jnp.float32),
        ],
    )


def kernel(topk_ids, topk_weights):
    t, k = topk_ids.shape
    ids_t, wout_t = _build(t, k)(topk_weights.T)
    return ids_t.T, wout_t.T
